# baseline (speedup 1.0000x reference)
import functools

import jax
import jax.numpy as jnp
from jax import lax
from jax.experimental import pallas as pl
from jax.experimental.pallas import tpu as pltpu
from jax.experimental.pallas import tpu_sc as plsc

D = 32
NTAB = 6
RCH = 4

REP_CAT = 16
REP_HOUR = 512
REP_DAY = 1024


def _build_sc_kernel(B: int, L: int, r_per_w: int, num_cores: int):
    n_ch = r_per_w // RCH
    assert r_per_w % RCH == 0 and n_ch % 2 == 0
    mesh = plsc.VectorSubcoreMesh(core_axis_name="c", subcore_axis_name="s")

    @functools.partial(
        pl.kernel,
        mesh=mesh,
        out_type=jax.ShapeDtypeStruct((B, L, NTAB, D), jnp.float32),
        compiler_params=pltpu.CompilerParams(use_tc_tiling_on_sc=False),
        scratch_types=[
            pltpu.VMEM((NTAB, r_per_w, L), jnp.int32),
            pltpu.VMEM((2, NTAB, RCH, L, D), jnp.float32),
            pltpu.SemaphoreType.DMA,
            pltpu.SemaphoreType.DMA,
            pltpu.SemaphoreType.DMA,
            pltpu.SemaphoreType.DMA,
        ],
    )
    def sc_kernel(u_t, p_t, c_t, h_t, d_t, q_t,
                  u_i, p_i, c_i, h_i, d_i, q_i,
                  out, idx_all, rows, sg0, sg1, sw0, sw1):
        wid = lax.axis_index("s") * num_cores + lax.axis_index("c")
        rbase = wid * r_per_w
        tabs = [u_t, p_t, c_t, h_t, d_t, q_t]
        idxs = [u_i, p_i, c_i, h_i, d_i, q_i]
        sg = [sg0, sg1]
        sw = [sw0, sw1]

        for t in range(NTAB):
            pltpu.sync_copy(idxs[t].at[pl.ds(rbase, r_per_w), :], idx_all.at[t])

        def gathers(ci, b):
            off = ci * RCH
            for t in range(NTAB):
                for r in range(RCH):
                    pltpu.async_copy(
                        tabs[t].at[idx_all.at[t, off + r]],
                        rows.at[b, t, r], sg[b])

        def wait_g(b):
            for t in range(NTAB):
                pltpu.make_async_copy(
                    out.at[pl.ds(0, RCH), :, 0, :],
                    rows.at[b, t], sg[b]).wait()

        def writes(ci, b):
            off = rbase + ci * RCH
            for t in range(NTAB):
                pltpu.async_copy(
                    rows.at[b, t],
                    out.at[pl.ds(off, RCH), :, t, :], sw[b])

        def wait_w(b):
            for t in range(NTAB):
                pltpu.make_async_copy(
                    rows.at[b, t],
                    out.at[pl.ds(0, RCH), :, 0, :], sw[b]).wait()

        gathers(0, 0)

        def body(i, carry):
            for k in range(2):
                ci = i * 2 + k
                b = k
                wait_g(b)
                writes(ci, b)

                @pl.when(ci + 1 < n_ch)
                def _issue_next():
                    @pl.when(ci > 0)
                    def _drain_writes():
                        wait_w(1 - b)
                    gathers(ci + 1, 1 - b)
            return carry

        lax.fori_loop(0, n_ch // 2, body, 0)
        wait_w(0)
        wait_w(1)

    return sc_kernel


def kernel(user_table, poi_table, cat_table, hour_table, day_table, qk_table,
           user_idx, poi_idx, category_idx, hour_idx, day_idx, quadkey_idx):
    B, L = user_idx.shape
    info = plsc.get_sparse_core_info()
    num_workers = info.num_cores * info.num_subcores
    r_per_w = B // num_workers

    spread = lax.broadcasted_iota(jnp.int32, (B, L), 1) + \
        lax.broadcasted_iota(jnp.int32, (B, L), 0)
    cat_rep = jnp.tile(cat_table, (REP_CAT, 1))
    hour_rep = jnp.tile(hour_table, (REP_HOUR, 1))
    day_rep = jnp.tile(day_table, (REP_DAY, 1))
    cat_i = category_idx + (spread % REP_CAT) * cat_table.shape[0]
    hour_i = hour_idx + (spread % REP_HOUR) * hour_table.shape[0]
    day_i = day_idx + (spread % REP_DAY) * day_table.shape[0]

    sc = _build_sc_kernel(B, L, r_per_w, info.num_cores)
    out = sc(user_table, poi_table, cat_rep, hour_rep, day_rep, qk_table,
             user_idx, poi_idx, cat_i, hour_i, day_i, quadkey_idx)
    return out.reshape(B, L, NTAB * D)


# speedup vs baseline: 1.3719x; 1.3719x over previous
import functools

import jax
import jax.numpy as jnp
from jax import lax
from jax.experimental import pallas as pl
from jax.experimental.pallas import tpu as pltpu
from jax.experimental.pallas import tpu_sc as plsc

D = 32
NTAB = 6
RCH = 4

REP_CAT = 16
REP_HOUR = 512
REP_DAY = 1024


def _build_sc_kernel(B: int, L: int, r_per_w: int, num_cores: int):
    n_ch = r_per_w // RCH
    assert r_per_w % RCH == 0 and n_ch % 2 == 0
    mesh = plsc.VectorSubcoreMesh(core_axis_name="c", subcore_axis_name="s")

    @functools.partial(
        pl.kernel,
        mesh=mesh,
        out_type=jax.ShapeDtypeStruct((B, L, NTAB * D), jnp.float32),
        compiler_params=pltpu.CompilerParams(use_tc_tiling_on_sc=False),
        scratch_types=[
            pltpu.VMEM((NTAB, r_per_w, L), jnp.int32),
            pltpu.VMEM((2, NTAB, RCH, L, D), jnp.float32),
            pltpu.SemaphoreType.DMA,
            pltpu.SemaphoreType.DMA,
            pltpu.SemaphoreType.DMA,
            pltpu.SemaphoreType.DMA,
        ],
    )
    def sc_kernel(u_t, p_t, c_t, h_t, d_t, q_t,
                  u_i, p_i, c_i, h_i, d_i, q_i,
                  out, idx_all, rows, sg0, sg1, sw0, sw1):
        wid = lax.axis_index("s") * num_cores + lax.axis_index("c")
        rbase = wid * r_per_w
        tabs = [u_t, p_t, c_t, h_t, d_t, q_t]
        idxs = [u_i, p_i, c_i, h_i, d_i, q_i]
        sg = [sg0, sg1]
        sw = [sw0, sw1]

        for t in range(NTAB):
            pltpu.sync_copy(idxs[t].at[pl.ds(rbase, r_per_w), :], idx_all.at[t])

        def gathers(ci, b):
            off = ci * RCH
            for t in range(NTAB):
                for r in range(RCH):
                    pltpu.async_copy(
                        tabs[t].at[idx_all.at[t, off + r]],
                        rows.at[b, t, r], sg[b])

        def wait_g(b):
            for t in range(NTAB):
                pltpu.make_async_copy(
                    out.at[pl.ds(0, RCH), :, pl.ds(0, D)],
                    rows.at[b, t], sg[b]).wait()

        def writes(ci, b):
            off = rbase + ci * RCH
            for t in range(NTAB):
                pltpu.async_copy(
                    rows.at[b, t],
                    out.at[pl.ds(off, RCH), :, pl.ds(t * D, D)], sw[b])

        def wait_w(b):
            for t in range(NTAB):
                pltpu.make_async_copy(
                    rows.at[b, t],
                    out.at[pl.ds(0, RCH), :, pl.ds(0, D)], sw[b]).wait()

        gathers(0, 0)

        def body(i, carry):
            for k in range(2):
                ci = i * 2 + k
                b = k
                wait_g(b)
                writes(ci, b)

                @pl.when(ci + 1 < n_ch)
                def _issue_next():
                    @pl.when(ci > 0)
                    def _drain_writes():
                        wait_w(1 - b)
                    gathers(ci + 1, 1 - b)
            return carry

        lax.fori_loop(0, n_ch // 2, body, 0)
        wait_w(0)
        wait_w(1)

    return sc_kernel


def kernel(user_table, poi_table, cat_table, hour_table, day_table, qk_table,
           user_idx, poi_idx, category_idx, hour_idx, day_idx, quadkey_idx):
    B, L = user_idx.shape
    info = plsc.get_sparse_core_info()
    num_workers = info.num_cores * info.num_subcores
    r_per_w = B // num_workers

    spread = lax.broadcasted_iota(jnp.int32, (B, L), 1) + \
        lax.broadcasted_iota(jnp.int32, (B, L), 0)
    cat_rep = jnp.tile(cat_table, (REP_CAT, 1))
    hour_rep = jnp.tile(hour_table, (REP_HOUR, 1))
    day_rep = jnp.tile(day_table, (REP_DAY, 1))
    cat_i = category_idx + (spread % REP_CAT) * cat_table.shape[0]
    hour_i = hour_idx + (spread % REP_HOUR) * hour_table.shape[0]
    day_i = day_idx + (spread % REP_DAY) * day_table.shape[0]

    sc = _build_sc_kernel(B, L, r_per_w, info.num_cores)
    return sc(user_table, poi_table, cat_rep, hour_rep, day_rep, qk_table,
              user_idx, poi_idx, cat_i, hour_i, day_i, quadkey_idx)
